# Initial kernel scaffold; baseline (speedup 1.0000x reference)
import jax
import jax.numpy as jnp
from jax.experimental import pallas as pl


def kernel(input, batch, weight, bias):
    raise NotImplementedError("write your pallas kernel here")


# speedup vs baseline: 1.1880x; 1.1880x over previous
import jax
import jax.numpy as jnp
from jax.experimental import pallas as pl

N = 100000
G = 512
EPS = 1e-05
BLK = 512
NB = 196
P = NB * BLK

_HI = jax.lax.Precision.HIGHEST


def _col_group(c):
    return jnp.where(c < 32, c, jnp.where(c < 128, 32 + (c - 32) // 3, 64 + (c - 128) // 5))


def _stats_kernel(x_ref, b_ref, out_ref):
    bidx = pl.program_id(0)
    x = x_ref[...]
    bb = b_ref[...]
    x2 = x * x
    c = jax.lax.broadcasted_iota(jnp.int32, (208, 80), 0)
    k = jax.lax.broadcasted_iota(jnp.int32, (208, 80), 1)
    M = (_col_group(c) == k).astype(jnp.float32)
    r2 = jnp.dot(x2, M, preferred_element_type=jnp.float32, precision=_HI)
    r = jnp.concatenate(
        [r2, x[:, :32], jnp.ones((BLK, 1), jnp.float32), jnp.zeros((BLK, 15), jnp.float32)],
        axis=1,
    )
    gi = jax.lax.broadcasted_iota(jnp.int32, (BLK, G), 1)
    oh = (bb == gi).astype(jnp.float32)
    contrib = jax.lax.dot_general(
        oh, r, (((0,), (0,)), ((), ())), preferred_element_type=jnp.float32, precision=_HI
    )

    @pl.when(bidx == 0)
    def _():
        out_ref[...] = jnp.zeros_like(out_ref)

    out_ref[...] += contrib


def _norm_kernel(s_ref, p_ref, t_ref):
    s = s_ref[...]
    w = p_ref[0:1, 0:80]
    bias = p_ref[1:2, 0:32]
    cnt = jnp.maximum(s[:, 112:113], 1.0)
    fm = s[:, 80:112] / cnt
    var0 = s[:, 0:32] / cnt - fm * fm
    fn1 = s[:, 32:64] / (3.0 * cnt)
    fn2 = s[:, 64:80] / (5.0 * cnt)
    fn = jnp.concatenate([var0, fn1, fn2], axis=1)
    inv = jax.lax.rsqrt(fn + EPS) * w
    k = jax.lax.broadcasted_iota(jnp.int32, (80, 208), 0)
    c = jax.lax.broadcasted_iota(jnp.int32, (80, 208), 1)
    MT = (_col_group(c) == k).astype(jnp.float32)
    A = jnp.dot(inv, MT, preferred_element_type=jnp.float32, precision=_HI)
    Bs = bias - fm * inv[:, 0:32]
    t_ref[...] = jnp.concatenate([A, Bs, jnp.zeros((G, 16), jnp.float32)], axis=1)


def _apply_kernel(x_ref, b_ref, t_ref, o_ref):
    x = x_ref[...]
    bb = b_ref[...]
    gi = jax.lax.broadcasted_iota(jnp.int32, (BLK, G), 1)
    oh = (bb == gi).astype(jnp.float32)
    t = t_ref[...]
    gath = jnp.dot(oh, t, preferred_element_type=jnp.float32, precision=_HI)
    o_ref[...] = x * gath[:, :208] + jnp.concatenate(
        [gath[:, 208:240], jnp.zeros((BLK, 176), jnp.float32)], axis=1
    )


def kernel(input, batch, weight, bias):
    x = jnp.pad(input, ((0, P - N), (0, 0)))
    b32 = jnp.pad(batch.astype(jnp.int32), (0, P - N), constant_values=G).reshape(P, 1)
    params = jnp.zeros((8, 128), jnp.float32).at[0, :80].set(weight).at[1, :32].set(bias)

    sums = pl.pallas_call(
        _stats_kernel,
        grid=(NB,),
        in_specs=[
            pl.BlockSpec((BLK, 208), lambda b: (b, 0)),
            pl.BlockSpec((BLK, 1), lambda b: (b, 0)),
        ],
        out_specs=pl.BlockSpec((G, 128), lambda b: (0, 0)),
        out_shape=jax.ShapeDtypeStruct((G, 128), jnp.float32),
    )(x, b32)

    table = pl.pallas_call(
        _norm_kernel,
        out_shape=jax.ShapeDtypeStruct((G, 256), jnp.float32),
    )(sums, params)

    out = pl.pallas_call(
        _apply_kernel,
        grid=(NB,),
        in_specs=[
            pl.BlockSpec((BLK, 208), lambda b: (b, 0)),
            pl.BlockSpec((BLK, 1), lambda b: (b, 0)),
            pl.BlockSpec((G, 256), lambda b: (0, 0)),
        ],
        out_specs=pl.BlockSpec((BLK, 208), lambda b: (b, 0)),
        out_shape=jax.ShapeDtypeStruct((P, 208), jnp.float32),
    )(x, b32, table)

    return out[:N]


# speedup vs baseline: 2.3707x; 1.9955x over previous
import jax
import jax.numpy as jnp
from jax.experimental import pallas as pl

N = 100000
G = 512
EPS = 1e-05
BLK = 1000
NB = 100

_HI = jax.lax.Precision.HIGHEST


def _col_group(c):
    return jnp.where(c < 32, c, jnp.where(c < 128, 32 + (c - 32) // 3, 64 + (c - 128) // 5))


def _stats_kernel(x_ref, b_ref, out_ref):
    bidx = pl.program_id(0)
    x = x_ref[...]
    bb = b_ref[...]
    x2 = x * x
    c = jax.lax.broadcasted_iota(jnp.int32, (208, 80), 0)
    k = jax.lax.broadcasted_iota(jnp.int32, (208, 80), 1)
    M = (_col_group(c) == k).astype(jnp.float32)
    r2 = jnp.dot(x2, M, preferred_element_type=jnp.float32, precision=_HI)
    r = jnp.concatenate(
        [r2, x[:, :32], jnp.ones((BLK, 1), jnp.float32), jnp.zeros((BLK, 15), jnp.float32)],
        axis=1,
    )
    gi = jax.lax.broadcasted_iota(jnp.int32, (BLK, G), 1)
    oh = (bb == gi).astype(jnp.float32)
    contrib = jax.lax.dot_general(
        oh, r, (((0,), (0,)), ((), ())), preferred_element_type=jnp.float32, precision=_HI
    )

    @pl.when(bidx == 0)
    def _():
        out_ref[...] = jnp.zeros_like(out_ref)

    out_ref[...] += contrib


def _norm_kernel(s_ref, p_ref, t_ref):
    s = s_ref[...]
    w = p_ref[0:1, 0:80]
    bias = p_ref[1:2, 0:32]
    cnt = jnp.maximum(s[:, 112:113], 1.0)
    fm = s[:, 80:112] / cnt
    var0 = s[:, 0:32] / cnt - fm * fm
    fn1 = s[:, 32:64] / (3.0 * cnt)
    fn2 = s[:, 64:80] / (5.0 * cnt)
    fn = jnp.concatenate([var0, fn1, fn2], axis=1)
    inv = jax.lax.rsqrt(fn + EPS) * w
    k = jax.lax.broadcasted_iota(jnp.int32, (80, 208), 0)
    c = jax.lax.broadcasted_iota(jnp.int32, (80, 208), 1)
    MT = (_col_group(c) == k).astype(jnp.float32)
    A = jnp.dot(inv, MT, preferred_element_type=jnp.float32, precision=_HI)
    Bs = bias - fm * inv[:, 0:32]
    t_ref[...] = jnp.concatenate([A, Bs, jnp.zeros((G, 16), jnp.float32)], axis=1)


def _apply_kernel(x_ref, b_ref, t_ref, o_ref):
    x = x_ref[...]
    bb = b_ref[...]
    gi = jax.lax.broadcasted_iota(jnp.int32, (BLK, G), 1)
    oh = (bb == gi).astype(jnp.float32)
    t = t_ref[...]
    gath = jnp.dot(oh, t, preferred_element_type=jnp.float32, precision=_HI)
    o_ref[...] = x * gath[:, :208] + jnp.concatenate(
        [gath[:, 208:240], jnp.zeros((BLK, 176), jnp.float32)], axis=1
    )


def kernel(input, batch, weight, bias):
    x = input
    b32 = batch.astype(jnp.int32).reshape(N, 1)
    params = jnp.zeros((8, 128), jnp.float32).at[0, :80].set(weight).at[1, :32].set(bias)

    sums = pl.pallas_call(
        _stats_kernel,
        grid=(NB,),
        in_specs=[
            pl.BlockSpec((BLK, 208), lambda b: (b, 0)),
            pl.BlockSpec((BLK, 1), lambda b: (b, 0)),
        ],
        out_specs=pl.BlockSpec((G, 128), lambda b: (0, 0)),
        out_shape=jax.ShapeDtypeStruct((G, 128), jnp.float32),
    )(x, b32)

    table = pl.pallas_call(
        _norm_kernel,
        out_shape=jax.ShapeDtypeStruct((G, 256), jnp.float32),
    )(sums, params)

    out = pl.pallas_call(
        _apply_kernel,
        grid=(NB,),
        in_specs=[
            pl.BlockSpec((BLK, 208), lambda b: (b, 0)),
            pl.BlockSpec((BLK, 1), lambda b: (b, 0)),
            pl.BlockSpec((G, 256), lambda b: (0, 0)),
        ],
        out_specs=pl.BlockSpec((BLK, 208), lambda b: (b, 0)),
        out_shape=jax.ShapeDtypeStruct((N, 208), jnp.float32),
    )(x, b32, table)

    return out


# speedup vs baseline: 4.2756x; 1.8035x over previous
import jax
import jax.numpy as jnp
from jax.experimental import pallas as pl

N = 100000
G = 512
EPS = 1e-05
BLK = 1000
NB = 100

_HI = jax.lax.Precision.HIGHEST


def _col_group(c):
    return jnp.where(c < 32, c, jnp.where(c < 128, 32 + (c - 32) // 3, 64 + (c - 128) // 5))


def _stats_kernel(x_ref, b_ref, out_ref):
    bidx = pl.program_id(0)
    x = x_ref[...]
    bb = b_ref[...]
    x2 = (x * x).astype(jnp.bfloat16)
    c = jax.lax.broadcasted_iota(jnp.int32, (208, 80), 0)
    k = jax.lax.broadcasted_iota(jnp.int32, (208, 80), 1)
    M = (_col_group(c) == k).astype(jnp.bfloat16)
    r2 = jnp.dot(x2, M, preferred_element_type=jnp.float32)
    r = jnp.concatenate(
        [r2, x[:, :32], jnp.ones((BLK, 1), jnp.float32), jnp.zeros((BLK, 15), jnp.float32)],
        axis=1,
    ).astype(jnp.bfloat16)
    gi = jax.lax.broadcasted_iota(jnp.int32, (BLK, G), 1)
    oh = (bb == gi).astype(jnp.bfloat16)
    contrib = jax.lax.dot_general(
        oh, r, (((0,), (0,)), ((), ())), preferred_element_type=jnp.float32
    )

    @pl.when(bidx == 0)
    def _():
        out_ref[...] = jnp.zeros_like(out_ref)

    out_ref[...] += contrib


def _norm_kernel(s_ref, p_ref, t_ref):
    s = s_ref[...]
    w = p_ref[0:1, 0:80]
    bias = p_ref[1:2, 0:32]
    cnt = jnp.maximum(s[:, 112:113], 1.0)
    fm = s[:, 80:112] / cnt
    var0 = s[:, 0:32] / cnt - fm * fm
    fn1 = s[:, 32:64] / (3.0 * cnt)
    fn2 = s[:, 64:80] / (5.0 * cnt)
    fn = jnp.concatenate([var0, fn1, fn2], axis=1)
    inv = jax.lax.rsqrt(fn + EPS) * w
    k = jax.lax.broadcasted_iota(jnp.int32, (80, 208), 0)
    c = jax.lax.broadcasted_iota(jnp.int32, (80, 208), 1)
    MT = (_col_group(c) == k).astype(jnp.float32)
    A = jnp.dot(inv, MT, preferred_element_type=jnp.float32, precision=_HI)
    Bs = bias - fm * inv[:, 0:32]
    t_ref[...] = jnp.concatenate([A, Bs, jnp.zeros((G, 16), jnp.float32)], axis=1)


def _apply_kernel(x_ref, b_ref, t_ref, o_ref):
    x = x_ref[...]
    bb = b_ref[...]
    gi = jax.lax.broadcasted_iota(jnp.int32, (BLK, G), 1)
    oh = (bb == gi).astype(jnp.bfloat16)
    t = t_ref[...]
    t_hi = t.astype(jnp.bfloat16)
    t_lo = (t - t_hi.astype(jnp.float32)).astype(jnp.bfloat16)
    gath = jnp.dot(oh, t_hi, preferred_element_type=jnp.float32) + jnp.dot(
        oh, t_lo, preferred_element_type=jnp.float32
    )
    o_ref[...] = x * gath[:, :208] + jnp.concatenate(
        [gath[:, 208:240], jnp.zeros((BLK, 176), jnp.float32)], axis=1
    )


def kernel(input, batch, weight, bias):
    x = input
    b32 = batch.astype(jnp.int32).reshape(N, 1)
    params = jnp.zeros((8, 128), jnp.float32).at[0, :80].set(weight).at[1, :32].set(bias)

    sums = pl.pallas_call(
        _stats_kernel,
        grid=(NB,),
        in_specs=[
            pl.BlockSpec((BLK, 208), lambda b: (b, 0)),
            pl.BlockSpec((BLK, 1), lambda b: (b, 0)),
        ],
        out_specs=pl.BlockSpec((G, 128), lambda b: (0, 0)),
        out_shape=jax.ShapeDtypeStruct((G, 128), jnp.float32),
    )(x, b32)

    table = pl.pallas_call(
        _norm_kernel,
        out_shape=jax.ShapeDtypeStruct((G, 256), jnp.float32),
    )(sums, params)

    out = pl.pallas_call(
        _apply_kernel,
        grid=(NB,),
        in_specs=[
            pl.BlockSpec((BLK, 208), lambda b: (b, 0)),
            pl.BlockSpec((BLK, 1), lambda b: (b, 0)),
            pl.BlockSpec((G, 256), lambda b: (0, 0)),
        ],
        out_specs=pl.BlockSpec((BLK, 208), lambda b: (b, 0)),
        out_shape=jax.ShapeDtypeStruct((N, 208), jnp.float32),
    )(x, b32, table)

    return out


# speedup vs baseline: 4.9794x; 1.1646x over previous
import jax
import jax.numpy as jnp
from jax.experimental import pallas as pl
from jax.experimental.pallas import tpu as pltpu

N = 100000
G = 512
EPS = 1e-05
BLK = 2000
NB = 50

_HI = jax.lax.Precision.HIGHEST


def _col_group(c):
    return jnp.where(c < 32, c, jnp.where(c < 128, 32 + (c - 32) // 3, 64 + (c - 128) // 5))


def _stats_kernel(x_ref, b_ref, p_ref, t_ref, acc_ref):
    bidx = pl.program_id(0)
    x = x_ref[...]
    bb = b_ref[...]
    x2 = (x * x).astype(jnp.bfloat16)
    c = jax.lax.broadcasted_iota(jnp.int32, (208, 80), 0)
    k = jax.lax.broadcasted_iota(jnp.int32, (208, 80), 1)
    M = (_col_group(c) == k).astype(jnp.bfloat16)
    r2 = jnp.dot(x2, M, preferred_element_type=jnp.float32)
    r = jnp.concatenate(
        [r2, x[:, :32], jnp.ones((BLK, 1), jnp.float32), jnp.zeros((BLK, 15), jnp.float32)],
        axis=1,
    ).astype(jnp.bfloat16)
    gi = jax.lax.broadcasted_iota(jnp.int32, (BLK, G), 1)
    oh = (bb == gi).astype(jnp.bfloat16)
    contrib = jax.lax.dot_general(
        oh, r, (((0,), (0,)), ((), ())), preferred_element_type=jnp.float32
    )

    @pl.when(bidx == 0)
    def _():
        acc_ref[...] = jnp.zeros_like(acc_ref)

    acc_ref[...] += contrib

    @pl.when(bidx == NB - 1)
    def _():
        s = acc_ref[...]
        w = p_ref[0:1, 0:80]
        bias = p_ref[1:2, 0:32]
        cnt = jnp.maximum(s[:, 112:113], 1.0)
        fm = s[:, 80:112] / cnt
        var0 = s[:, 0:32] / cnt - fm * fm
        fn1 = s[:, 32:64] / (3.0 * cnt)
        fn2 = s[:, 64:80] / (5.0 * cnt)
        fn = jnp.concatenate([var0, fn1, fn2], axis=1)
        inv = jax.lax.rsqrt(fn + EPS) * w
        kk = jax.lax.broadcasted_iota(jnp.int32, (80, 208), 0)
        cc = jax.lax.broadcasted_iota(jnp.int32, (80, 208), 1)
        MT = (_col_group(cc) == kk).astype(jnp.float32)
        A = jnp.dot(inv, MT, preferred_element_type=jnp.float32, precision=_HI)
        Bs = bias - fm * inv[:, 0:32]
        t_ref[...] = jnp.concatenate([A, Bs, jnp.zeros((G, 16), jnp.float32)], axis=1)


def _apply_kernel(x_ref, b_ref, t_ref, o_ref):
    x = x_ref[...]
    bb = b_ref[...]
    gi = jax.lax.broadcasted_iota(jnp.int32, (BLK, G), 1)
    oh = (bb == gi).astype(jnp.bfloat16)
    t = t_ref[...]
    t_hi = t.astype(jnp.bfloat16)
    t_lo = (t - t_hi.astype(jnp.float32)).astype(jnp.bfloat16)
    gath = jnp.dot(oh, t_hi, preferred_element_type=jnp.float32) + jnp.dot(
        oh, t_lo, preferred_element_type=jnp.float32
    )
    o_ref[...] = x * gath[:, :208] + jnp.concatenate(
        [gath[:, 208:240], jnp.zeros((BLK, 176), jnp.float32)], axis=1
    )


def kernel(input, batch, weight, bias):
    x = input
    b32 = batch.astype(jnp.int32).reshape(N, 1)
    params = jnp.zeros((8, 128), jnp.float32).at[0, :80].set(weight).at[1, :32].set(bias)

    table = pl.pallas_call(
        _stats_kernel,
        grid=(NB,),
        in_specs=[
            pl.BlockSpec((BLK, 208), lambda b: (b, 0)),
            pl.BlockSpec((BLK, 1), lambda b: (b, 0)),
            pl.BlockSpec((8, 128), lambda b: (0, 0)),
        ],
        out_specs=pl.BlockSpec((G, 256), lambda b: (0, 0)),
        out_shape=jax.ShapeDtypeStruct((G, 256), jnp.float32),
        scratch_shapes=[pltpu.VMEM((G, 128), jnp.float32)],
    )(x, b32, params)

    out = pl.pallas_call(
        _apply_kernel,
        grid=(NB,),
        in_specs=[
            pl.BlockSpec((BLK, 208), lambda b: (b, 0)),
            pl.BlockSpec((BLK, 1), lambda b: (b, 0)),
            pl.BlockSpec((G, 256), lambda b: (0, 0)),
        ],
        out_specs=pl.BlockSpec((BLK, 208), lambda b: (b, 0)),
        out_shape=jax.ShapeDtypeStruct((N, 208), jnp.float32),
    )(x, b32, table)

    return out


# speedup vs baseline: 5.4525x; 1.0950x over previous
import jax
import jax.numpy as jnp
from jax.experimental import pallas as pl
from jax.experimental.pallas import tpu as pltpu

N = 100000
G = 512
EPS = 1e-05
BLK = 5000
NB = 20

_HI = jax.lax.Precision.HIGHEST


def _col_group(c):
    return jnp.where(c < 32, c, jnp.where(c < 128, 32 + (c - 32) // 3, 64 + (c - 128) // 5))


def _stats_kernel(x_ref, b_ref, p_ref, t_ref, acc_ref):
    bidx = pl.program_id(0)
    x = x_ref[...]
    bb = b_ref[...]
    x2 = (x * x).astype(jnp.bfloat16)
    c = jax.lax.broadcasted_iota(jnp.int32, (208, 80), 0)
    k = jax.lax.broadcasted_iota(jnp.int32, (208, 80), 1)
    M = (_col_group(c) == k).astype(jnp.bfloat16)
    r2 = jnp.dot(x2, M, preferred_element_type=jnp.float32)
    r = jnp.concatenate(
        [r2, x[:, :32], jnp.ones((BLK, 1), jnp.float32), jnp.zeros((BLK, 15), jnp.float32)],
        axis=1,
    ).astype(jnp.bfloat16)
    gi = jax.lax.broadcasted_iota(jnp.int32, (BLK, G), 1)
    oh = (bb == gi).astype(jnp.bfloat16)
    contrib = jax.lax.dot_general(
        oh, r, (((0,), (0,)), ((), ())), preferred_element_type=jnp.float32
    )

    @pl.when(bidx == 0)
    def _():
        acc_ref[...] = jnp.zeros_like(acc_ref)

    acc_ref[...] += contrib

    @pl.when(bidx == NB - 1)
    def _():
        s = acc_ref[...]
        w = p_ref[0:1, 0:80]
        bias = p_ref[1:2, 0:32]
        cnt = jnp.maximum(s[:, 112:113], 1.0)
        fm = s[:, 80:112] / cnt
        var0 = s[:, 0:32] / cnt - fm * fm
        fn1 = s[:, 32:64] / (3.0 * cnt)
        fn2 = s[:, 64:80] / (5.0 * cnt)
        fn = jnp.concatenate([var0, fn1, fn2], axis=1)
        inv = jax.lax.rsqrt(fn + EPS) * w
        kk = jax.lax.broadcasted_iota(jnp.int32, (80, 208), 0)
        cc = jax.lax.broadcasted_iota(jnp.int32, (80, 208), 1)
        MT = (_col_group(cc) == kk).astype(jnp.float32)
        A = jnp.dot(inv, MT, preferred_element_type=jnp.float32, precision=_HI)
        Bs = bias - fm * inv[:, 0:32]
        t_ref[...] = jnp.concatenate([A, Bs, jnp.zeros((G, 16), jnp.float32)], axis=1)


def _apply_kernel(x_ref, b_ref, t_ref, o_ref):
    x = x_ref[...]
    bb = b_ref[...]
    gi = jax.lax.broadcasted_iota(jnp.int32, (BLK, G), 1)
    oh = (bb == gi).astype(jnp.bfloat16)
    t = t_ref[...]
    t_hi = t.astype(jnp.bfloat16)
    t_lo = (t - t_hi.astype(jnp.float32)).astype(jnp.bfloat16)
    gath = jnp.dot(oh, t_hi, preferred_element_type=jnp.float32) + jnp.dot(
        oh, t_lo, preferred_element_type=jnp.float32
    )
    o_ref[...] = x * gath[:, :208] + jnp.concatenate(
        [gath[:, 208:240], jnp.zeros((BLK, 176), jnp.float32)], axis=1
    )


def kernel(input, batch, weight, bias):
    x = input
    b32 = batch.astype(jnp.int32).reshape(N, 1)
    params = jnp.zeros((8, 128), jnp.float32).at[0, :80].set(weight).at[1, :32].set(bias)

    table = pl.pallas_call(
        _stats_kernel,
        grid=(NB,),
        in_specs=[
            pl.BlockSpec((BLK, 208), lambda b: (b, 0)),
            pl.BlockSpec((BLK, 1), lambda b: (b, 0)),
            pl.BlockSpec((8, 128), lambda b: (0, 0)),
        ],
        out_specs=pl.BlockSpec((G, 256), lambda b: (0, 0)),
        out_shape=jax.ShapeDtypeStruct((G, 256), jnp.float32),
        scratch_shapes=[pltpu.VMEM((G, 128), jnp.float32)],
    )(x, b32, params)

    out = pl.pallas_call(
        _apply_kernel,
        grid=(NB,),
        in_specs=[
            pl.BlockSpec((BLK, 208), lambda b: (b, 0)),
            pl.BlockSpec((BLK, 1), lambda b: (b, 0)),
            pl.BlockSpec((G, 256), lambda b: (0, 0)),
        ],
        out_specs=pl.BlockSpec((BLK, 208), lambda b: (b, 0)),
        out_shape=jax.ShapeDtypeStruct((N, 208), jnp.float32),
    )(x, b32, table)

    return out


# speedup vs baseline: 5.5091x; 1.0104x over previous
import jax
import jax.numpy as jnp
from jax.experimental import pallas as pl
from jax.experimental.pallas import tpu as pltpu

N = 100000
G = 512
EPS = 1e-05
BLKA = 10000
NBA = 10
BLK = 5000
NB = 20

_HI = jax.lax.Precision.HIGHEST


def _col_group(c):
    return jnp.where(c < 32, c, jnp.where(c < 128, 32 + (c - 32) // 3, 64 + (c - 128) // 5))


def _stats_kernel(x_ref, b_ref, p_ref, t_ref, acc_ref):
    bidx = pl.program_id(0)
    x = x_ref[...]
    bb = b_ref[...]
    x2 = (x * x).astype(jnp.bfloat16)
    c = jax.lax.broadcasted_iota(jnp.int32, (208, 80), 0)
    k = jax.lax.broadcasted_iota(jnp.int32, (208, 80), 1)
    M = (_col_group(c) == k).astype(jnp.bfloat16)
    r2 = jnp.dot(x2, M, preferred_element_type=jnp.float32)
    r = jnp.concatenate(
        [r2, x[:, :32], jnp.ones((BLKA, 1), jnp.float32), jnp.zeros((BLKA, 15), jnp.float32)],
        axis=1,
    ).astype(jnp.bfloat16)
    gi = jax.lax.broadcasted_iota(jnp.int32, (BLKA, G), 1)
    oh = (bb == gi).astype(jnp.bfloat16)
    contrib = jax.lax.dot_general(
        oh, r, (((0,), (0,)), ((), ())), preferred_element_type=jnp.float32
    )

    @pl.when(bidx == 0)
    def _():
        acc_ref[...] = jnp.zeros_like(acc_ref)

    acc_ref[...] += contrib

    @pl.when(bidx == NBA - 1)
    def _():
        s = acc_ref[...]
        w = p_ref[0:1, 0:80]
        bias = p_ref[1:2, 0:32]
        cnt = jnp.maximum(s[:, 112:113], 1.0)
        fm = s[:, 80:112] / cnt
        var0 = s[:, 0:32] / cnt - fm * fm
        fn1 = s[:, 32:64] / (3.0 * cnt)
        fn2 = s[:, 64:80] / (5.0 * cnt)
        fn = jnp.concatenate([var0, fn1, fn2], axis=1)
        inv = jax.lax.rsqrt(fn + EPS) * w
        kk = jax.lax.broadcasted_iota(jnp.int32, (80, 208), 0)
        cc = jax.lax.broadcasted_iota(jnp.int32, (80, 208), 1)
        MT = (_col_group(cc) == kk).astype(jnp.float32)
        A = jnp.dot(inv, MT, preferred_element_type=jnp.float32, precision=_HI)
        Bs = bias - fm * inv[:, 0:32]
        t_ref[...] = jnp.concatenate([A, Bs, jnp.zeros((G, 16), jnp.float32)], axis=1)


def _apply_kernel(x_ref, b_ref, t_ref, o_ref):
    x = x_ref[...]
    bb = b_ref[...]
    gi = jax.lax.broadcasted_iota(jnp.int32, (BLK, G), 1)
    oh = (bb == gi).astype(jnp.bfloat16)
    t = t_ref[...]
    t_hi = t.astype(jnp.bfloat16)
    t_lo = (t - t_hi.astype(jnp.float32)).astype(jnp.bfloat16)
    gath = jnp.dot(oh, t_hi, preferred_element_type=jnp.float32) + jnp.dot(
        oh, t_lo, preferred_element_type=jnp.float32
    )
    o_ref[...] = x * gath[:, :208] + jnp.concatenate(
        [gath[:, 208:240], jnp.zeros((BLK, 176), jnp.float32)], axis=1
    )


def kernel(input, batch, weight, bias):
    x = input
    b32 = batch.astype(jnp.int32).reshape(N, 1)
    params = jnp.zeros((8, 128), jnp.float32).at[0, :80].set(weight).at[1, :32].set(bias)

    table = pl.pallas_call(
        _stats_kernel,
        grid=(NBA,),
        in_specs=[
            pl.BlockSpec((BLKA, 208), lambda b: (b, 0)),
            pl.BlockSpec((BLKA, 1), lambda b: (b, 0)),
            pl.BlockSpec((8, 128), lambda b: (0, 0)),
        ],
        out_specs=pl.BlockSpec((G, 256), lambda b: (0, 0)),
        out_shape=jax.ShapeDtypeStruct((G, 256), jnp.float32),
        scratch_shapes=[pltpu.VMEM((G, 128), jnp.float32)],
    )(x, b32, params)

    out = pl.pallas_call(
        _apply_kernel,
        grid=(NB,),
        in_specs=[
            pl.BlockSpec((BLK, 208), lambda b: (b, 0)),
            pl.BlockSpec((BLK, 1), lambda b: (b, 0)),
            pl.BlockSpec((G, 256), lambda b: (0, 0)),
        ],
        out_specs=pl.BlockSpec((BLK, 208), lambda b: (b, 0)),
        out_shape=jax.ShapeDtypeStruct((N, 208), jnp.float32),
    )(x, b32, table)

    return out


# speedup vs baseline: 6.4803x; 1.1763x over previous
import jax
import jax.numpy as jnp
from jax.experimental import pallas as pl
from jax.experimental.pallas import tpu as pltpu

N = 100000
G = 512
EPS = 1e-05
BLKA = 10000
NBA = 10
BLK = 5000
NB = 20

_HI = jax.lax.Precision.HIGHEST


def _col_group(c):
    return jnp.where(c < 32, c, jnp.where(c < 128, 32 + (c - 32) // 3, 64 + (c - 128) // 5))


def _stats_kernel(x_ref, b_ref, p_ref, t_ref, acc_ref):
    bidx = pl.program_id(0)
    x = x_ref[...]
    bb = b_ref[...].reshape(1, BLKA)
    x2 = (x * x).astype(jnp.bfloat16)
    c = jax.lax.broadcasted_iota(jnp.int32, (208, 80), 0)
    k = jax.lax.broadcasted_iota(jnp.int32, (208, 80), 1)
    M = (_col_group(c) == k).astype(jnp.bfloat16)
    r2 = jnp.dot(x2, M, preferred_element_type=jnp.float32)
    r = jnp.concatenate(
        [r2, x[:, :32], jnp.ones((BLKA, 1), jnp.float32), jnp.zeros((BLKA, 15), jnp.float32)],
        axis=1,
    ).astype(jnp.bfloat16)
    gi = jax.lax.broadcasted_iota(jnp.int32, (G, BLKA), 0)
    oht = (bb == gi).astype(jnp.bfloat16)
    contrib = jax.lax.dot_general(
        oht, r, (((1,), (0,)), ((), ())), preferred_element_type=jnp.float32
    )

    @pl.when(bidx == 0)
    def _():
        acc_ref[...] = jnp.zeros_like(acc_ref)

    acc_ref[...] += contrib

    @pl.when(bidx == NBA - 1)
    def _():
        s = acc_ref[...]
        w = p_ref[0:1, 0:80]
        bias = p_ref[1:2, 0:32]
        cnt = jnp.maximum(s[:, 112:113], 1.0)
        fm = s[:, 80:112] / cnt
        var0 = s[:, 0:32] / cnt - fm * fm
        fn1 = s[:, 32:64] / (3.0 * cnt)
        fn2 = s[:, 64:80] / (5.0 * cnt)
        fn = jnp.concatenate([var0, fn1, fn2], axis=1)
        inv = jax.lax.rsqrt(fn + EPS) * w
        kk = jax.lax.broadcasted_iota(jnp.int32, (80, 208), 0)
        cc = jax.lax.broadcasted_iota(jnp.int32, (80, 208), 1)
        MT = (_col_group(cc) == kk).astype(jnp.float32)
        A = jnp.dot(inv, MT, preferred_element_type=jnp.float32, precision=_HI)
        Bs = bias - fm * inv[:, 0:32]
        t_ref[...] = jnp.concatenate([A, Bs, jnp.zeros((G, 16), jnp.float32)], axis=1)


def _apply_kernel(x_ref, b_ref, t_ref, o_ref):
    x = x_ref[...]
    bb = b_ref[...].reshape(1, BLK)
    gi = jax.lax.broadcasted_iota(jnp.int32, (G, BLK), 0)
    oht = (bb == gi).astype(jnp.bfloat16)
    t = t_ref[...]
    t_hi = t.astype(jnp.bfloat16)
    t_lo = (t - t_hi.astype(jnp.float32)).astype(jnp.bfloat16)
    gath = jax.lax.dot_general(
        oht, t_hi, (((0,), (0,)), ((), ())), preferred_element_type=jnp.float32
    ) + jax.lax.dot_general(
        oht, t_lo, (((0,), (0,)), ((), ())), preferred_element_type=jnp.float32
    )
    o_ref[...] = x * gath[:, :208] + jnp.concatenate(
        [gath[:, 208:240], jnp.zeros((BLK, 176), jnp.float32)], axis=1
    )


def kernel(input, batch, weight, bias):
    x = input
    b32 = batch.astype(jnp.int32)
    ba = b32.reshape(NBA, 1, BLKA)
    bc = b32.reshape(NB, 1, BLK)
    params = jnp.zeros((8, 128), jnp.float32).at[0, :80].set(weight).at[1, :32].set(bias)

    table = pl.pallas_call(
        _stats_kernel,
        grid=(NBA,),
        in_specs=[
            pl.BlockSpec((BLKA, 208), lambda b: (b, 0)),
            pl.BlockSpec((1, 1, BLKA), lambda b: (b, 0, 0)),
            pl.BlockSpec((8, 128), lambda b: (0, 0)),
        ],
        out_specs=pl.BlockSpec((G, 256), lambda b: (0, 0)),
        out_shape=jax.ShapeDtypeStruct((G, 256), jnp.float32),
        scratch_shapes=[pltpu.VMEM((G, 128), jnp.float32)],
    )(x, ba, params)

    out = pl.pallas_call(
        _apply_kernel,
        grid=(NB,),
        in_specs=[
            pl.BlockSpec((BLK, 208), lambda b: (b, 0)),
            pl.BlockSpec((1, 1, BLK), lambda b: (b, 0, 0)),
            pl.BlockSpec((G, 256), lambda b: (0, 0)),
        ],
        out_specs=pl.BlockSpec((BLK, 208), lambda b: (b, 0)),
        out_shape=jax.ShapeDtypeStruct((N, 208), jnp.float32),
    )(x, bc, table)

    return out


# speedup vs baseline: 6.6321x; 1.0234x over previous
import jax
import jax.numpy as jnp
from jax.experimental import pallas as pl
from jax.experimental.pallas import tpu as pltpu

N = 100000
G = 512
EPS = 1e-05
BLKA = 10000
NBA = 10
BLK = 5000
NB = 20

_HI = jax.lax.Precision.HIGHEST


def _col_group(c):
    return jnp.where(c < 32, c, jnp.where(c < 128, 32 + (c - 32) // 3, 64 + (c - 128) // 5))


def _stats_kernel(x_ref, b_ref, p_ref, t_ref, acc_ref):
    bidx = pl.program_id(0)
    x = x_ref[...]
    bb = b_ref[...].reshape(1, BLKA)
    x2 = (x * x).astype(jnp.bfloat16)
    c = jax.lax.broadcasted_iota(jnp.int32, (208, 80), 0)
    k = jax.lax.broadcasted_iota(jnp.int32, (208, 80), 1)
    M = (_col_group(c) == k).astype(jnp.bfloat16)
    r2 = jnp.dot(x2, M, preferred_element_type=jnp.float32)
    r = jnp.concatenate(
        [r2, x[:, :32], jnp.ones((BLKA, 1), jnp.float32), jnp.zeros((BLKA, 15), jnp.float32)],
        axis=1,
    ).astype(jnp.bfloat16)
    gi = jax.lax.broadcasted_iota(jnp.int32, (G, BLKA), 0)
    oht = (bb == gi).astype(jnp.bfloat16)
    contrib = jax.lax.dot_general(
        oht, r, (((1,), (0,)), ((), ())), preferred_element_type=jnp.float32
    )

    @pl.when(bidx == 0)
    def _():
        acc_ref[...] = jnp.zeros_like(acc_ref)

    acc_ref[...] += contrib

    @pl.when(bidx == NBA - 1)
    def _():
        s = acc_ref[...]
        w = p_ref[0:1, 0:80]
        bias = p_ref[1:2, 0:32]
        cnt = jnp.maximum(s[:, 112:113], 1.0)
        fm = s[:, 80:112] / cnt
        var0 = s[:, 0:32] / cnt - fm * fm
        fn1 = s[:, 32:64] / (3.0 * cnt)
        fn2 = s[:, 64:80] / (5.0 * cnt)
        fn = jnp.concatenate([var0, fn1, fn2], axis=1)
        inv = jax.lax.rsqrt(fn + EPS) * w
        kk = jax.lax.broadcasted_iota(jnp.int32, (80, 208), 0)
        cc = jax.lax.broadcasted_iota(jnp.int32, (80, 208), 1)
        MT = (_col_group(cc) == kk).astype(jnp.float32)
        A = jnp.dot(inv, MT, preferred_element_type=jnp.float32, precision=_HI)
        Bs = bias - fm * inv[:, 0:32]
        t_ref[...] = jnp.concatenate([A, Bs, jnp.zeros((G, 16), jnp.float32)], axis=1)


def _apply_kernel(x_ref, b_ref, t_ref, o_ref):
    x = x_ref[...]
    bb = b_ref[...].reshape(1, BLK)
    gi = jax.lax.broadcasted_iota(jnp.int32, (G, BLK), 0)
    oht = (bb == gi).astype(jnp.bfloat16)
    t = t_ref[...]
    t_hi = t.astype(jnp.bfloat16)
    gath = jax.lax.dot_general(
        oht, t_hi, (((0,), (0,)), ((), ())), preferred_element_type=jnp.float32
    )
    o_ref[...] = x * gath[:, :208] + jnp.concatenate(
        [gath[:, 208:240], jnp.zeros((BLK, 176), jnp.float32)], axis=1
    )


def kernel(input, batch, weight, bias):
    x = input
    b32 = batch.astype(jnp.int32)
    ba = b32.reshape(NBA, 1, BLKA)
    bc = b32.reshape(NB, 1, BLK)
    params = jnp.zeros((8, 128), jnp.float32).at[0, :80].set(weight).at[1, :32].set(bias)

    table = pl.pallas_call(
        _stats_kernel,
        grid=(NBA,),
        in_specs=[
            pl.BlockSpec((BLKA, 208), lambda b: (b, 0)),
            pl.BlockSpec((1, 1, BLKA), lambda b: (b, 0, 0)),
            pl.BlockSpec((8, 128), lambda b: (0, 0)),
        ],
        out_specs=pl.BlockSpec((G, 256), lambda b: (0, 0)),
        out_shape=jax.ShapeDtypeStruct((G, 256), jnp.float32),
        scratch_shapes=[pltpu.VMEM((G, 128), jnp.float32)],
    )(x, ba, params)

    out = pl.pallas_call(
        _apply_kernel,
        grid=(NB,),
        in_specs=[
            pl.BlockSpec((BLK, 208), lambda b: (b, 0)),
            pl.BlockSpec((1, 1, BLK), lambda b: (b, 0, 0)),
            pl.BlockSpec((G, 256), lambda b: (0, 0)),
        ],
        out_specs=pl.BlockSpec((BLK, 208), lambda b: (b, 0)),
        out_shape=jax.ShapeDtypeStruct((N, 208), jnp.float32),
    )(x, bc, table)

    return out
